# baseline (speedup 1.0000x reference)
import functools

import jax
import jax.numpy as jnp
from jax import lax
from jax.experimental import pallas as pl
from jax.experimental.pallas import tpu as pltpu
from jax.experimental.pallas import tpu_sc as plsc

NUM_VOCAB = 1000000
D = 64
PD = 128
BATCH = 4096
SEQ = 200
LANES = 16

NC = 2
NS = 16
NW = NC * NS

BW = BATCH // NW
NB = 4
NO = 2
NJ = BW // LANES


def _emb_body(x_hbm, emb_hbm, pos_hbm, out_hbm,
              xb_v, rows_v, outb_v, pos_s, xq_v, par_s, gsem, osem):
    wid = lax.axis_index("s") * NC + lax.axis_index("c")
    col = wid * BW

    pltpu.sync_copy(x_hbm.at[:, pl.ds(col, BW)], xb_v)
    pltpu.sync_copy(pos_hbm, pos_s)
    iota = lax.iota(jnp.int32, LANES)

    def gather_start(l, b):
        for j0 in range(NJ):
            sl = pl.ds(j0 * LANES, LANES)
            xq_v[b, sl] = lax.shift_right_logical(xb_v[l, sl], 1)
        pltpu.async_copy(emb_hbm.at[xq_v.at[b]], rows_v.at[b], gsem.at[b])

    def gather_wait(l, b):
        pltpu.make_async_copy(emb_hbm.at[xq_v.at[b]], rows_v.at[b],
                              gsem.at[b]).wait()

    def out_start(l, o):
        pltpu.async_copy(outb_v.at[o], out_hbm.at[l, :, pl.ds(col, BW)],
                         osem.at[o])

    def out_wait(l, o):
        pltpu.make_async_copy(outb_v.at[o], out_hbm.at[l, :, pl.ds(col, BW)],
                              osem.at[o]).wait()

    def compute(l, b, o):
        rows = rows_v.at[b]
        lvec = jnp.full((LANES,), 0, jnp.int32) + l
        for j0 in range(NJ):
            sl = pl.ds(j0 * LANES, LANES)
            par_s[sl] = lax.shift_left(
                jnp.bitwise_and(xb_v[l, sl], 1), 6)

        def c_body(c, _):
            cvec = jnp.full((LANES,), 0, jnp.int32) + c
            p = plsc.load_gather(pos_s, [cvec, lvec])
            for j0 in range(NJ):
                sl = pl.ds(j0 * LANES, LANES)
                jvec = iota + (j0 * LANES)
                pc = par_s[sl] + cvec
                v = plsc.load_gather(rows, [jvec, pc])
                outb_v[o, c, sl] = v + p
            return _
        lax.fori_loop(0, D, c_body, 0)

    for l in range(NB - 1):
        gather_start(l, l)

    def group_body(grp, carry):
        for u in range(NB):
            l = grp * NB + u
            b = u

            @pl.when(l + NB - 1 < SEQ)
            def _():
                gather_start(l + NB - 1, (l + NB - 1) % NB)

            gather_wait(l, b)
            o = l % NO

            @pl.when(l >= NO)
            def _():
                out_wait(l - NO, o)

            compute(l, b, o)
            out_start(l, o)
        return carry

    lax.fori_loop(0, SEQ // NB, group_body, 0)

    for u in range(NO):
        out_wait(SEQ - NO + u, u)


@jax.jit
def kernel(x, emb_table, pos_table):
    x_t = x.T.astype(jnp.int32)
    emb2 = emb_table.reshape(NUM_VOCAB // 2, 2 * D)
    pos_pad = jnp.pad(pos_table.T[:, :SEQ], ((0, 0), (0, 256 - SEQ)))

    mesh = plsc.VectorSubcoreMesh(core_axis_name="c", subcore_axis_name="s")
    run = pl.kernel(
        _emb_body,
        mesh=mesh,
        out_type=jax.ShapeDtypeStruct((SEQ, D, BATCH), jnp.float32),
        compiler_params=pltpu.CompilerParams(needs_layout_passes=False),
        scratch_types=[
            pltpu.VMEM((SEQ, BW), jnp.int32),
            pltpu.VMEM((NB, BW, PD), jnp.float32),
            pltpu.VMEM((NO, D, BW), jnp.float32),
            pltpu.VMEM((D, 256), jnp.float32),
            pltpu.VMEM((NB, BW), jnp.int32),
            pltpu.VMEM((BW,), jnp.int32),
            pltpu.SemaphoreType.DMA((NB,)),
            pltpu.SemaphoreType.DMA((NO,)),
        ],
    )
    out = run(x_t, emb2, pos_pad)
    return out.transpose(2, 0, 1)


# speedup vs baseline: 2.1407x; 2.1407x over previous
import functools

import jax
import jax.numpy as jnp
from jax import lax
from jax.experimental import pallas as pl
from jax.experimental.pallas import tpu as pltpu
from jax.experimental.pallas import tpu_sc as plsc

NUM_VOCAB = 1000000
MAXLEN = 200
D = 64
BATCH = 4096
SEQ = 200

NC = 2
NS = 16
NW = NC * NS

B_TOTAL = BATCH * SEQ
ROWS_PER_W = B_TOTAL // NW
CH = SEQ
NSTEP = ROWS_PER_W // CH
NBUF = 4
SEQ_PER_W = BATCH // NW


def _emb_body(x_hbm, emb_hbm, pos_hbm, out_hbm,
              idx_v, pos_v, rows_v, gsem, osem):
    wid = lax.axis_index("s") * NC + lax.axis_index("c")
    my_base = wid * ROWS_PER_W
    my_seq = wid * SEQ_PER_W

    pltpu.sync_copy(x_hbm.at[pl.ds(my_base, ROWS_PER_W)], idx_v)
    pltpu.sync_copy(pos_hbm.at[pl.ds(0, SEQ)], pos_v)

    def gather_start(g, b):
        idx = idx_v.at[pl.ds(g * CH, CH)]
        pltpu.async_copy(emb_hbm.at[idx], rows_v.at[b], gsem.at[b])

    def gather_wait(g, b):
        idx = idx_v.at[pl.ds(g * CH, CH)]
        pltpu.make_async_copy(emb_hbm.at[idx], rows_v.at[b], gsem.at[b]).wait()

    def out_start(g, b):
        pltpu.async_copy(rows_v.at[b], out_hbm.at[my_seq + g], osem.at[b])

    def out_wait(g, b):
        pltpu.make_async_copy(rows_v.at[b], out_hbm.at[my_seq + g],
                              osem.at[b]).wait()

    for b in range(NBUF - 1):
        gather_start(b, b)

    def group_body(grp, carry):
        for b in range(NBUF):
            g = grp * NBUF + b
            gather_wait(g, b)

            def add_row(r, c):
                for j in range(D // 16):
                    sl = pl.ds(j * 16, 16)
                    rows_v[b, r, sl] = rows_v[b, r, sl] + pos_v[r, sl]
                return c
            lax.fori_loop(0, CH, add_row, 0)

            out_start(g, b)

            b2 = (g + NBUF - 1) % NBUF

            @pl.when(g + NBUF - 1 < NSTEP)
            def _():
                @pl.when(g > 0)
                def _():
                    out_wait(g - 1, b2)
                gather_start(g + NBUF - 1, b2)
        return carry

    lax.fori_loop(0, NSTEP // NBUF, group_body, 0)

    for b in range(NBUF):
        g = NSTEP - NBUF + b
        out_wait(g, b)


@jax.jit
def kernel(x, emb_table, pos_table):
    x_flat = x.reshape(-1).astype(jnp.int32)

    mesh = plsc.VectorSubcoreMesh(core_axis_name="c", subcore_axis_name="s")
    run = pl.kernel(
        _emb_body,
        mesh=mesh,
        out_type=jax.ShapeDtypeStruct((BATCH, SEQ, D), jnp.float32),
        compiler_params=pltpu.CompilerParams(use_tc_tiling_on_sc=False),
        scratch_types=[
            pltpu.VMEM((ROWS_PER_W,), jnp.int32),
            pltpu.VMEM((SEQ, D), jnp.float32),
            pltpu.VMEM((NBUF, CH, D), jnp.float32),
            pltpu.SemaphoreType.DMA((NBUF,)),
            pltpu.SemaphoreType.DMA((NBUF,)),
        ],
    )
    return run(x_flat, emb_table, pos_table)
